# baseline (speedup 1.0000x reference)
import functools

import jax
import jax.numpy as jnp
from jax import lax
from jax.experimental import pallas as pl
from jax.experimental.pallas import tpu as pltpu
from jax.experimental.pallas import tpu_sc as plsc

N_NODES = 10000
N_EDGES = 320000
NTILES = 16
CHUNK = 128
CH_PER_TILE = 160
IDXBLK = 16
EDGES_PER_TILE = CH_PER_TILE * CHUNK
E_PAD = EDGES_PER_TILE * NTILES
N_PAD = 10112
ROWS_PER_TILE = N_PAD // NTILES
ACC_ROWS = N_PAD + 16
RBLK = 1264
NUM_GRAPHS = 64
DH = 128


def _make_sc_agg():
    mesh = plsc.VectorSubcoreMesh(core_axis_name="c", subcore_axis_name="s")

    @functools.partial(
        pl.kernel,
        out_type=[
            jax.ShapeDtypeStruct((N_PAD, DH), jnp.float32),
            jax.ShapeDtypeStruct((N_PAD, DH), jnp.float32),
        ],
        mesh=mesh,
        scratch_types=[
            pltpu.VMEM((IDXBLK, CHUNK), jnp.int32),
            pltpu.VMEM((IDXBLK, CHUNK), jnp.int32),
            pltpu.VMEM((CHUNK, DH), jnp.float32),
            pltpu.VMEM((CHUNK, DH), jnp.float32),
            pltpu.VMEM_SHARED((ACC_ROWS, DH), jnp.float32),
            pltpu.SemaphoreType.DMA,
            pltpu.SemaphoreType.DMA,
            pltpu.SemaphoreType.DMA,
            pltpu.SemaphoreType.DMA,
        ],
    )
    def sc_agg(pt, pb, srcs, dsts, qt, qb,
               src_v, dst_v, rows0, rows1, acc, gsem0, gsem1, ssem0, ssem1):
        c = lax.axis_index("c")
        s = lax.axis_index("s")

        def run(p_hbm, q_hbm):
            row0 = s * ROWS_PER_TILE
            pltpu.sync_copy(p_hbm.at[pl.ds(row0, ROWS_PER_TILE)],
                            acc.at[pl.ds(row0, ROWS_PER_TILE)])
            plsc.subcore_barrier()

            bufs = (rows0, rows1)
            gsems = (gsem0, gsem1)
            ssems = (ssem0, ssem1)

            def block(ib, carry):
                pltpu.sync_copy(srcs.at[s, pl.ds(ib * IDXBLK, IDXBLK)], src_v)
                pltpu.sync_copy(dsts.at[s, pl.ds(ib * IDXBLK, IDXBLK)], dst_v)
                gd = [pltpu.async_copy(p_hbm.at[src_v.at[0]], rows0, gsem0),
                      pltpu.async_copy(p_hbm.at[src_v.at[1]], rows1, gsem1)]
                for b in range(IDXBLK):
                    par = b % 2
                    gd[par].wait()
                    pltpu.async_copy(bufs[par], acc.at[dst_v.at[b]],
                                     ssems[par], add=True).wait()
                    if b + 2 < IDXBLK:
                        gd[par] = pltpu.async_copy(
                            p_hbm.at[src_v.at[b + 2]], bufs[par], gsems[par])
                return carry

            lax.fori_loop(0, CH_PER_TILE // IDXBLK, block, 0)
            plsc.subcore_barrier()
            pltpu.sync_copy(acc.at[pl.ds(row0, ROWS_PER_TILE)],
                            q_hbm.at[pl.ds(row0, ROWS_PER_TILE)])

        @pl.when(c == 0)
        def _():
            run(pt, qt)

        @pl.when(c == 1)
        def _():
            run(pb, qb)

    return sc_agg


@functools.lru_cache(maxsize=None)
def _make_tc_matmul(dh_in, dout, with_act):
    grid = (N_PAD // RBLK,)

    def body(qt_ref, qb_ref, b_ref, w_ref, yt_ref, yb_ref):
        a = jnp.concatenate([qt_ref[...], qb_ref[...]], axis=1)
        if with_act:
            a = jnp.maximum(a + b_ref[...], 0.0)
        y = jnp.dot(a, w_ref[...], preferred_element_type=jnp.float32)
        yt_ref[...] = y[:, :dout // 2]
        yb_ref[...] = y[:, dout // 2:]

    return pl.pallas_call(
        body,
        grid=grid,
        in_specs=[
            pl.BlockSpec((RBLK, dh_in), lambda i: (i, 0)),
            pl.BlockSpec((RBLK, dh_in), lambda i: (i, 0)),
            pl.BlockSpec((1, 2 * dh_in), lambda i: (0, 0)),
            pl.BlockSpec((2 * dh_in, dout), lambda i: (0, 0)),
        ],
        out_specs=[
            pl.BlockSpec((RBLK, dout // 2), lambda i: (i, 0)),
            pl.BlockSpec((RBLK, dout // 2), lambda i: (i, 0)),
        ],
        out_shape=[
            jax.ShapeDtypeStruct((N_PAD, dout // 2), jnp.float32),
            jax.ShapeDtypeStruct((N_PAD, dout // 2), jnp.float32),
        ],
    )


def _make_pool_fc():
    grid = (N_PAD // RBLK,)

    def body(bt_ref, qt_ref, qb_ref, b3_ref, wf1_ref, bf1_ref, wf2_ref,
             bf2_ref, out_ref, sums_ref, cnts_ref):
        i = pl.program_id(0)

        @pl.when(i == 0)
        def _():
            sums_ref[...] = jnp.zeros_like(sums_ref)
            cnts_ref[...] = jnp.zeros_like(cnts_ref)

        seg_ids = bt_ref[0, 0, :]
        seg = (seg_ids[None, :]
               == lax.broadcasted_iota(jnp.int32, (NUM_GRAPHS, RBLK), 0)
               ).astype(jnp.float32)
        q = jnp.concatenate([qt_ref[...], qb_ref[...]], axis=1)
        y = jnp.maximum(q + b3_ref[...], 0.0)
        sums_ref[...] += jnp.dot(seg, y, preferred_element_type=jnp.float32)
        cnts_ref[...] += jnp.broadcast_to(
            jnp.sum(seg, axis=1, keepdims=True), cnts_ref.shape)

        @pl.when(i == pl.num_programs(0) - 1)
        def _():
            cnt = cnts_ref[...][:, 0:1]
            hg = sums_ref[...] / jnp.maximum(cnt, 1.0)
            h1 = jnp.maximum(
                jnp.dot(hg, wf1_ref[...], preferred_element_type=jnp.float32)
                + bf1_ref[...], 0.0)
            out_ref[...] = (
                jnp.dot(h1, wf2_ref[...], preferred_element_type=jnp.float32)
                + bf2_ref[...])

    return pl.pallas_call(
        body,
        grid=grid,
        in_specs=[
            pl.BlockSpec((1, 1, RBLK), lambda i: (i, 0, 0)),
            pl.BlockSpec((RBLK, 128), lambda i: (i, 0)),
            pl.BlockSpec((RBLK, 128), lambda i: (i, 0)),
            pl.BlockSpec((1, 256), lambda i: (0, 0)),
            pl.BlockSpec((256, 128), lambda i: (0, 0)),
            pl.BlockSpec((1, 128), lambda i: (0, 0)),
            pl.BlockSpec((128, 64), lambda i: (0, 0)),
            pl.BlockSpec((1, 64), lambda i: (0, 0)),
        ],
        out_specs=pl.BlockSpec((NUM_GRAPHS, 64), lambda i: (0, 0)),
        out_shape=jax.ShapeDtypeStruct((NUM_GRAPHS, 64), jnp.float32),
        scratch_shapes=[
            pltpu.VMEM((NUM_GRAPHS, 256), jnp.float32),
            pltpu.VMEM((NUM_GRAPHS, 128), jnp.float32),
        ],
        compiler_params=pltpu.CompilerParams(
            dimension_semantics=("arbitrary",)),
    )


def kernel(x, edge_index, batch, W1, b1, W2, b2, W3, b3, Wf1, bf1, Wf2, bf2):
    src = edge_index[0].astype(jnp.int32)
    dst = edge_index[1].astype(jnp.int32)
    pad = E_PAD - N_EDGES
    srcs = jnp.concatenate([src, jnp.zeros((pad,), jnp.int32)]).reshape(
        NTILES, CH_PER_TILE, CHUNK)
    dsts = jnp.concatenate([dst, jnp.full((pad,), N_PAD, jnp.int32)]
                           ).reshape(NTILES, CH_PER_TILE, CHUNK)

    xp = jnp.pad(x, ((0, N_PAD - N_NODES), (0, 0)))
    xt, xb = xp[:, :64], xp[:, 64:]
    sc_agg = _make_sc_agg()

    pt, pb = _make_tc_matmul(64, 256, False)(xt, xb,
                                             jnp.zeros((1, 128), jnp.float32),
                                             W1)
    qt, qb = sc_agg(pt, pb, srcs, dsts)
    pt, pb = _make_tc_matmul(128, 256, True)(qt, qb, b1.reshape(1, 256), W2)
    qt, qb = sc_agg(pt, pb, srcs, dsts)
    pt, pb = _make_tc_matmul(128, 256, True)(qt, qb, b2.reshape(1, 256), W3)
    qt, qb = sc_agg(pt, pb, srcs, dsts)

    batch_p = jnp.pad(batch.astype(jnp.int32), (0, N_PAD - N_NODES),
                      constant_values=NUM_GRAPHS)
    batch_r = batch_p.reshape(N_PAD // RBLK, 1, RBLK)
    out = _make_pool_fc()(batch_r, qt, qb, b3.reshape(1, 256),
                          Wf1, bf1.reshape(1, 128), Wf2, bf2.reshape(1, 64))
    return out


# speedup vs baseline: 1.0178x; 1.0178x over previous
import functools

import jax
import jax.numpy as jnp
from jax import lax
from jax.experimental import pallas as pl
from jax.experimental.pallas import tpu as pltpu
from jax.experimental.pallas import tpu_sc as plsc

N_NODES = 10000
N_EDGES = 320000
NTILES = 16
CHUNK = 128
CH_PER_TILE = 160
IDXBLK = 32
EDGES_PER_TILE = CH_PER_TILE * CHUNK
E_PAD = EDGES_PER_TILE * NTILES
N_PAD = 10112
ROWS_PER_TILE = N_PAD // NTILES
ACC_ROWS = N_PAD + 16
RBLK = 1264
NUM_GRAPHS = 64
DH = 128


def _make_sc_agg():
    mesh = plsc.VectorSubcoreMesh(core_axis_name="c", subcore_axis_name="s")

    @functools.partial(
        pl.kernel,
        out_type=[
            jax.ShapeDtypeStruct((N_PAD, DH), jnp.float32),
            jax.ShapeDtypeStruct((N_PAD, DH), jnp.float32),
        ],
        mesh=mesh,
        scratch_types=[
            pltpu.VMEM((IDXBLK, CHUNK), jnp.int32),
            pltpu.VMEM((IDXBLK, CHUNK), jnp.int32),
            pltpu.VMEM((CHUNK, DH), jnp.float32),
            pltpu.VMEM((CHUNK, DH), jnp.float32),
            pltpu.VMEM_SHARED((ACC_ROWS, DH), jnp.float32),
            pltpu.SemaphoreType.DMA,
            pltpu.SemaphoreType.DMA,
            pltpu.SemaphoreType.DMA,
            pltpu.SemaphoreType.DMA,
        ],
    )
    def sc_agg(pt, pb, srcs, dsts, qt, qb,
               src_v, dst_v, rows0, rows1, acc, gsem0, gsem1, ssem0, ssem1):
        c = lax.axis_index("c")
        s = lax.axis_index("s")

        def run(p_hbm, q_hbm):
            row0 = s * ROWS_PER_TILE
            pltpu.sync_copy(p_hbm.at[pl.ds(row0, ROWS_PER_TILE)],
                            acc.at[pl.ds(row0, ROWS_PER_TILE)])
            plsc.subcore_barrier()

            bufs = (rows0, rows1)
            gsems = (gsem0, gsem1)
            ssems = (ssem0, ssem1)

            def block(ib, carry):
                pltpu.sync_copy(srcs.at[s, pl.ds(ib * IDXBLK, IDXBLK)], src_v)
                pltpu.sync_copy(dsts.at[s, pl.ds(ib * IDXBLK, IDXBLK)], dst_v)
                gd = [pltpu.async_copy(p_hbm.at[src_v.at[0]], rows0, gsem0),
                      pltpu.async_copy(p_hbm.at[src_v.at[1]], rows1, gsem1)]
                for b in range(IDXBLK):
                    par = b % 2
                    gd[par].wait()
                    pltpu.async_copy(bufs[par], acc.at[dst_v.at[b]],
                                     ssems[par], add=True).wait()
                    if b + 2 < IDXBLK:
                        gd[par] = pltpu.async_copy(
                            p_hbm.at[src_v.at[b + 2]], bufs[par], gsems[par])
                return carry

            lax.fori_loop(0, CH_PER_TILE // IDXBLK, block, 0)
            plsc.subcore_barrier()
            pltpu.sync_copy(acc.at[pl.ds(row0, ROWS_PER_TILE)],
                            q_hbm.at[pl.ds(row0, ROWS_PER_TILE)])

        @pl.when(c == 0)
        def _():
            run(pt, qt)

        @pl.when(c == 1)
        def _():
            run(pb, qb)

    return sc_agg


@functools.lru_cache(maxsize=None)
def _make_tc_matmul(dh_in, dout, with_act):
    grid = (N_PAD // RBLK,)

    def body(qt_ref, qb_ref, b_ref, w_ref, yt_ref, yb_ref):
        a = jnp.concatenate([qt_ref[...], qb_ref[...]], axis=1)
        if with_act:
            a = jnp.maximum(a + b_ref[...], 0.0)
        y = jnp.dot(a, w_ref[...], preferred_element_type=jnp.float32)
        yt_ref[...] = y[:, :dout // 2]
        yb_ref[...] = y[:, dout // 2:]

    return pl.pallas_call(
        body,
        grid=grid,
        in_specs=[
            pl.BlockSpec((RBLK, dh_in), lambda i: (i, 0)),
            pl.BlockSpec((RBLK, dh_in), lambda i: (i, 0)),
            pl.BlockSpec((1, 2 * dh_in), lambda i: (0, 0)),
            pl.BlockSpec((2 * dh_in, dout), lambda i: (0, 0)),
        ],
        out_specs=[
            pl.BlockSpec((RBLK, dout // 2), lambda i: (i, 0)),
            pl.BlockSpec((RBLK, dout // 2), lambda i: (i, 0)),
        ],
        out_shape=[
            jax.ShapeDtypeStruct((N_PAD, dout // 2), jnp.float32),
            jax.ShapeDtypeStruct((N_PAD, dout // 2), jnp.float32),
        ],
    )


def _make_pool_fc():
    grid = (N_PAD // RBLK,)

    def body(bt_ref, qt_ref, qb_ref, b3_ref, wf1_ref, bf1_ref, wf2_ref,
             bf2_ref, out_ref, sums_ref, cnts_ref):
        i = pl.program_id(0)

        @pl.when(i == 0)
        def _():
            sums_ref[...] = jnp.zeros_like(sums_ref)
            cnts_ref[...] = jnp.zeros_like(cnts_ref)

        seg_ids = bt_ref[0, 0, :]
        seg = (seg_ids[None, :]
               == lax.broadcasted_iota(jnp.int32, (NUM_GRAPHS, RBLK), 0)
               ).astype(jnp.float32)
        q = jnp.concatenate([qt_ref[...], qb_ref[...]], axis=1)
        y = jnp.maximum(q + b3_ref[...], 0.0)
        sums_ref[...] += jnp.dot(seg, y, preferred_element_type=jnp.float32)
        cnts_ref[...] += jnp.broadcast_to(
            jnp.sum(seg, axis=1, keepdims=True), cnts_ref.shape)

        @pl.when(i == pl.num_programs(0) - 1)
        def _():
            cnt = cnts_ref[...][:, 0:1]
            hg = sums_ref[...] / jnp.maximum(cnt, 1.0)
            h1 = jnp.maximum(
                jnp.dot(hg, wf1_ref[...], preferred_element_type=jnp.float32)
                + bf1_ref[...], 0.0)
            out_ref[...] = (
                jnp.dot(h1, wf2_ref[...], preferred_element_type=jnp.float32)
                + bf2_ref[...])

    return pl.pallas_call(
        body,
        grid=grid,
        in_specs=[
            pl.BlockSpec((1, 1, RBLK), lambda i: (i, 0, 0)),
            pl.BlockSpec((RBLK, 128), lambda i: (i, 0)),
            pl.BlockSpec((RBLK, 128), lambda i: (i, 0)),
            pl.BlockSpec((1, 256), lambda i: (0, 0)),
            pl.BlockSpec((256, 128), lambda i: (0, 0)),
            pl.BlockSpec((1, 128), lambda i: (0, 0)),
            pl.BlockSpec((128, 64), lambda i: (0, 0)),
            pl.BlockSpec((1, 64), lambda i: (0, 0)),
        ],
        out_specs=pl.BlockSpec((NUM_GRAPHS, 64), lambda i: (0, 0)),
        out_shape=jax.ShapeDtypeStruct((NUM_GRAPHS, 64), jnp.float32),
        scratch_shapes=[
            pltpu.VMEM((NUM_GRAPHS, 256), jnp.float32),
            pltpu.VMEM((NUM_GRAPHS, 128), jnp.float32),
        ],
        compiler_params=pltpu.CompilerParams(
            dimension_semantics=("arbitrary",)),
    )


def kernel(x, edge_index, batch, W1, b1, W2, b2, W3, b3, Wf1, bf1, Wf2, bf2):
    src = edge_index[0].astype(jnp.int32)
    dst = edge_index[1].astype(jnp.int32)
    pad = E_PAD - N_EDGES
    srcs = jnp.concatenate([src, jnp.zeros((pad,), jnp.int32)]).reshape(
        NTILES, CH_PER_TILE, CHUNK)
    dsts = jnp.concatenate([dst, jnp.full((pad,), N_PAD, jnp.int32)]
                           ).reshape(NTILES, CH_PER_TILE, CHUNK)

    xp = jnp.pad(x, ((0, N_PAD - N_NODES), (0, 0)))
    xt, xb = xp[:, :64], xp[:, 64:]
    sc_agg = _make_sc_agg()

    pt, pb = _make_tc_matmul(64, 256, False)(xt, xb,
                                             jnp.zeros((1, 128), jnp.float32),
                                             W1)
    qt, qb = sc_agg(pt, pb, srcs, dsts)
    pt, pb = _make_tc_matmul(128, 256, True)(qt, qb, b1.reshape(1, 256), W2)
    qt, qb = sc_agg(pt, pb, srcs, dsts)
    pt, pb = _make_tc_matmul(128, 256, True)(qt, qb, b2.reshape(1, 256), W3)
    qt, qb = sc_agg(pt, pb, srcs, dsts)

    batch_p = jnp.pad(batch.astype(jnp.int32), (0, N_PAD - N_NODES),
                      constant_values=NUM_GRAPHS)
    batch_r = batch_p.reshape(N_PAD // RBLK, 1, RBLK)
    out = _make_pool_fc()(batch_r, qt, qb, b3.reshape(1, 256),
                          Wf1, bf1.reshape(1, 128), Wf2, bf2.reshape(1, 64))
    return out


# speedup vs baseline: 1.0197x; 1.0019x over previous
import functools

import jax
import jax.numpy as jnp
from jax import lax
from jax.experimental import pallas as pl
from jax.experimental.pallas import tpu as pltpu
from jax.experimental.pallas import tpu_sc as plsc

N_NODES = 10000
N_EDGES = 320000
NTILES = 16
CHUNK = 128
CH_PER_TILE = 160
IDXBLK = 40
EDGES_PER_TILE = CH_PER_TILE * CHUNK
E_PAD = EDGES_PER_TILE * NTILES
N_PAD = 10112
ROWS_PER_TILE = N_PAD // NTILES
ACC_ROWS = N_PAD + 16
RBLK = 1264
NUM_GRAPHS = 64
DH = 128


def _make_sc_agg():
    mesh = plsc.VectorSubcoreMesh(core_axis_name="c", subcore_axis_name="s")

    @functools.partial(
        pl.kernel,
        out_type=[
            jax.ShapeDtypeStruct((N_PAD, DH), jnp.float32),
            jax.ShapeDtypeStruct((N_PAD, DH), jnp.float32),
        ],
        mesh=mesh,
        scratch_types=[
            pltpu.VMEM((IDXBLK, CHUNK), jnp.int32),
            pltpu.VMEM((IDXBLK, CHUNK), jnp.int32),
            pltpu.VMEM((CHUNK, DH), jnp.float32),
            pltpu.VMEM((CHUNK, DH), jnp.float32),
            pltpu.VMEM_SHARED((ACC_ROWS, DH), jnp.float32),
            pltpu.SemaphoreType.DMA,
            pltpu.SemaphoreType.DMA,
            pltpu.SemaphoreType.DMA,
            pltpu.SemaphoreType.DMA,
        ],
    )
    def sc_agg(pt, pb, srcs, dsts, qt, qb,
               src_v, dst_v, rows0, rows1, acc, gsem0, gsem1, ssem0, ssem1):
        c = lax.axis_index("c")
        s = lax.axis_index("s")

        def run(p_hbm, q_hbm):
            row0 = s * ROWS_PER_TILE
            pltpu.sync_copy(p_hbm.at[pl.ds(row0, ROWS_PER_TILE)],
                            acc.at[pl.ds(row0, ROWS_PER_TILE)])
            plsc.subcore_barrier()

            bufs = (rows0, rows1)
            gsems = (gsem0, gsem1)
            ssems = (ssem0, ssem1)

            def block(ib, carry):
                pltpu.sync_copy(srcs.at[s, pl.ds(ib * IDXBLK, IDXBLK)], src_v)
                pltpu.sync_copy(dsts.at[s, pl.ds(ib * IDXBLK, IDXBLK)], dst_v)
                gd = [pltpu.async_copy(p_hbm.at[src_v.at[0]], rows0, gsem0),
                      pltpu.async_copy(p_hbm.at[src_v.at[1]], rows1, gsem1)]
                for b in range(IDXBLK):
                    par = b % 2
                    gd[par].wait()
                    pltpu.async_copy(bufs[par], acc.at[dst_v.at[b]],
                                     ssems[par], add=True).wait()
                    if b + 2 < IDXBLK:
                        gd[par] = pltpu.async_copy(
                            p_hbm.at[src_v.at[b + 2]], bufs[par], gsems[par])
                return carry

            lax.fori_loop(0, CH_PER_TILE // IDXBLK, block, 0)
            plsc.subcore_barrier()
            pltpu.sync_copy(acc.at[pl.ds(row0, ROWS_PER_TILE)],
                            q_hbm.at[pl.ds(row0, ROWS_PER_TILE)])

        @pl.when(c == 0)
        def _():
            run(pt, qt)

        @pl.when(c == 1)
        def _():
            run(pb, qb)

    return sc_agg


@functools.lru_cache(maxsize=None)
def _make_tc_matmul(dh_in, dout, with_act):
    grid = (N_PAD // RBLK,)

    def body(qt_ref, qb_ref, b_ref, w_ref, yt_ref, yb_ref):
        a = jnp.concatenate([qt_ref[...], qb_ref[...]], axis=1)
        if with_act:
            a = jnp.maximum(a + b_ref[...], 0.0)
        y = jnp.dot(a, w_ref[...], preferred_element_type=jnp.float32)
        yt_ref[...] = y[:, :dout // 2]
        yb_ref[...] = y[:, dout // 2:]

    return pl.pallas_call(
        body,
        grid=grid,
        in_specs=[
            pl.BlockSpec((RBLK, dh_in), lambda i: (i, 0)),
            pl.BlockSpec((RBLK, dh_in), lambda i: (i, 0)),
            pl.BlockSpec((1, 2 * dh_in), lambda i: (0, 0)),
            pl.BlockSpec((2 * dh_in, dout), lambda i: (0, 0)),
        ],
        out_specs=[
            pl.BlockSpec((RBLK, dout // 2), lambda i: (i, 0)),
            pl.BlockSpec((RBLK, dout // 2), lambda i: (i, 0)),
        ],
        out_shape=[
            jax.ShapeDtypeStruct((N_PAD, dout // 2), jnp.float32),
            jax.ShapeDtypeStruct((N_PAD, dout // 2), jnp.float32),
        ],
    )


def _make_pool_fc():
    grid = (N_PAD // RBLK,)

    def body(bt_ref, qt_ref, qb_ref, b3_ref, wf1_ref, bf1_ref, wf2_ref,
             bf2_ref, out_ref, sums_ref, cnts_ref):
        i = pl.program_id(0)

        @pl.when(i == 0)
        def _():
            sums_ref[...] = jnp.zeros_like(sums_ref)
            cnts_ref[...] = jnp.zeros_like(cnts_ref)

        seg_ids = bt_ref[0, 0, :]
        seg = (seg_ids[None, :]
               == lax.broadcasted_iota(jnp.int32, (NUM_GRAPHS, RBLK), 0)
               ).astype(jnp.float32)
        q = jnp.concatenate([qt_ref[...], qb_ref[...]], axis=1)
        y = jnp.maximum(q + b3_ref[...], 0.0)
        sums_ref[...] += jnp.dot(seg, y, preferred_element_type=jnp.float32)
        cnts_ref[...] += jnp.broadcast_to(
            jnp.sum(seg, axis=1, keepdims=True), cnts_ref.shape)

        @pl.when(i == pl.num_programs(0) - 1)
        def _():
            cnt = cnts_ref[...][:, 0:1]
            hg = sums_ref[...] / jnp.maximum(cnt, 1.0)
            h1 = jnp.maximum(
                jnp.dot(hg, wf1_ref[...], preferred_element_type=jnp.float32)
                + bf1_ref[...], 0.0)
            out_ref[...] = (
                jnp.dot(h1, wf2_ref[...], preferred_element_type=jnp.float32)
                + bf2_ref[...])

    return pl.pallas_call(
        body,
        grid=grid,
        in_specs=[
            pl.BlockSpec((1, 1, RBLK), lambda i: (i, 0, 0)),
            pl.BlockSpec((RBLK, 128), lambda i: (i, 0)),
            pl.BlockSpec((RBLK, 128), lambda i: (i, 0)),
            pl.BlockSpec((1, 256), lambda i: (0, 0)),
            pl.BlockSpec((256, 128), lambda i: (0, 0)),
            pl.BlockSpec((1, 128), lambda i: (0, 0)),
            pl.BlockSpec((128, 64), lambda i: (0, 0)),
            pl.BlockSpec((1, 64), lambda i: (0, 0)),
        ],
        out_specs=pl.BlockSpec((NUM_GRAPHS, 64), lambda i: (0, 0)),
        out_shape=jax.ShapeDtypeStruct((NUM_GRAPHS, 64), jnp.float32),
        scratch_shapes=[
            pltpu.VMEM((NUM_GRAPHS, 256), jnp.float32),
            pltpu.VMEM((NUM_GRAPHS, 128), jnp.float32),
        ],
        compiler_params=pltpu.CompilerParams(
            dimension_semantics=("arbitrary",)),
    )


def kernel(x, edge_index, batch, W1, b1, W2, b2, W3, b3, Wf1, bf1, Wf2, bf2):
    src = edge_index[0].astype(jnp.int32)
    dst = edge_index[1].astype(jnp.int32)
    pad = E_PAD - N_EDGES
    srcs = jnp.concatenate([src, jnp.zeros((pad,), jnp.int32)]).reshape(
        NTILES, CH_PER_TILE, CHUNK)
    dsts = jnp.concatenate([dst, jnp.full((pad,), N_PAD, jnp.int32)]
                           ).reshape(NTILES, CH_PER_TILE, CHUNK)

    xp = jnp.pad(x, ((0, N_PAD - N_NODES), (0, 0)))
    xt, xb = xp[:, :64], xp[:, 64:]
    sc_agg = _make_sc_agg()

    pt, pb = _make_tc_matmul(64, 256, False)(xt, xb,
                                             jnp.zeros((1, 128), jnp.float32),
                                             W1)
    qt, qb = sc_agg(pt, pb, srcs, dsts)
    pt, pb = _make_tc_matmul(128, 256, True)(qt, qb, b1.reshape(1, 256), W2)
    qt, qb = sc_agg(pt, pb, srcs, dsts)
    pt, pb = _make_tc_matmul(128, 256, True)(qt, qb, b2.reshape(1, 256), W3)
    qt, qb = sc_agg(pt, pb, srcs, dsts)

    batch_p = jnp.pad(batch.astype(jnp.int32), (0, N_PAD - N_NODES),
                      constant_values=NUM_GRAPHS)
    batch_r = batch_p.reshape(N_PAD // RBLK, 1, RBLK)
    out = _make_pool_fc()(batch_r, qt, qb, b3.reshape(1, 256),
                          Wf1, bf1.reshape(1, 128), Wf2, bf2.reshape(1, 64))
    return out
